# Initial kernel scaffold; baseline (speedup 1.0000x reference)
import jax
import jax.numpy as jnp
from jax.experimental import pallas as pl


def kernel(features, image, bn_gamma, bn_beta, bn_mean, bn_var, conv_w, conv_b):
    raise NotImplementedError("write your pallas kernel here")


# speedup vs baseline: 1.3014x; 1.3014x over previous
import functools

import jax
import jax.numpy as jnp
from jax.experimental import pallas as pl
from jax.experimental.pallas import tpu as pltpu


def _round_up(x, m):
    return ((x + m - 1) // m) * m


def _interp_matrix_t(out_size, in_size):
    scale = in_size / out_size
    i = jnp.arange(out_size, dtype=jnp.float32)
    src = jnp.maximum((i + 0.5) * scale - 0.5, 0.0)
    i0 = jnp.minimum(jnp.floor(src).astype(jnp.int32), in_size - 1)
    i1 = jnp.minimum(i0 + 1, in_size - 1)
    frac = src - i0.astype(jnp.float32)
    w = (jax.nn.one_hot(i0, in_size, dtype=jnp.float32) * (1.0 - frac)[:, None]
         + jax.nn.one_hot(i1, in_size, dtype=jnp.float32) * frac[:, None])
    return w.T


def _logits_kernel(x_ref, wt_ref, b_ref, o_ref):
    x = x_ref[0, 1:, :]
    y = jax.lax.dot_general(
        wt_ref[...], x, (((1,), (1,)), ((), ())),
        preferred_element_type=jnp.float32)
    y = y + b_ref[...]
    o_ref[0] = y[:o_ref.shape[1], :]


def _upsample_kernel(p_ref, wct_ref, wr_ref, o_ref):
    G = o_ref.shape[0]
    P = p_ref.shape[1]
    u = jnp.dot(p_ref[...], wct_ref[...],
                preferred_element_type=jnp.float32)
    u = u.astype(jnp.bfloat16).reshape(G, P, 256)
    wr = wr_ref[...]
    for g in range(G):
        o_ref[g] = jnp.dot(wr, u[g],
                           preferred_element_type=jnp.float32)[:, :o_ref.shape[2]]


def _pick_group(total, cap=16):
    g, best_even = 1, 0
    for cand in range(1, cap + 1):
        if total % cand == 0:
            g = cand
            if (total // cand) % 2 == 0:
                best_even = cand
    return best_even or g


@functools.partial(jax.jit, static_argnames=())
def kernel(features, image, bn_gamma, bn_beta, bn_mean, bn_var, conv_w, conv_b):
    B, n_tok, E = features.shape
    img = image.shape[-1]
    P = img // 14
    assert n_tok == 1 + P * P
    C = conv_w.shape[0]
    Cpad = _round_up(C, 8)

    inv_std = 1.0 / jnp.sqrt(bn_var + 1e-5)
    scale = bn_gamma * inv_std
    shift = bn_beta - bn_mean * scale
    w2 = conv_w.reshape(C, E)
    wt = jnp.zeros((Cpad, E), jnp.float32).at[:C].set(w2 * scale[None, :])
    b2 = jnp.zeros((Cpad, 1), jnp.float32).at[:C, 0].set(conv_b + w2 @ shift)

    planes = pl.pallas_call(
        _logits_kernel,
        out_shape=jax.ShapeDtypeStruct((B, C, P * P), jnp.float32),
        grid=(B,),
        in_specs=[
            pl.BlockSpec((1, n_tok, E), lambda b: (b, 0, 0)),
            pl.BlockSpec((Cpad, E), lambda b: (0, 0)),
            pl.BlockSpec((Cpad, 1), lambda b: (0, 0)),
        ],
        out_specs=pl.BlockSpec((1, C, P * P), lambda b: (b, 0, 0)),
        compiler_params=pltpu.CompilerParams(dimension_semantics=("parallel",)),
        cost_estimate=pl.CostEstimate(
            flops=2 * B * P * P * E * Cpad, transcendentals=0,
            bytes_accessed=4 * (B * n_tok * E + B * C * P * P + Cpad * E)),
    )(features, wt, b2)

    planes_r = planes.reshape(B * C * P, P)

    wrt = _interp_matrix_t(img, P)
    wct = jnp.zeros((P, 256), jnp.float32).at[:, :img].set(wrt)
    wr = wrt.T.astype(jnp.bfloat16)

    NP = B * C
    G = _pick_group(NP)
    up = pl.pallas_call(
        _upsample_kernel,
        out_shape=jax.ShapeDtypeStruct((NP, img, img), jnp.float32),
        grid=(NP // G,),
        in_specs=[
            pl.BlockSpec((G * P, P), lambda n: (n, 0)),
            pl.BlockSpec((P, 256), lambda n: (0, 0)),
            pl.BlockSpec((img, P), lambda n: (0, 0)),
        ],
        out_specs=pl.BlockSpec((G, img, img), lambda n: (n, 0, 0)),
        compiler_params=pltpu.CompilerParams(dimension_semantics=("parallel",)),
        cost_estimate=pl.CostEstimate(
            flops=2 * NP * (P * P * 256 + P * img * 256), transcendentals=0,
            bytes_accessed=4 * (NP * P * P + NP * img * img)),
    )(planes_r, wct, wr)

    return up.reshape(B, C, img, img)
